# baseline (speedup 1.0000x reference)
import jax
import jax.numpy as jnp
from jax.experimental import pallas as pl
from jax.experimental.pallas import tpu as pltpu


def _copy_body(se_in, rw_in, se_out, rw_out, sem1, sem2):
    rows = se_out.shape[0]
    c1 = pltpu.make_async_copy(se_in.at[pl.ds(0, rows)], se_out, sem1)
    c2 = pltpu.make_async_copy(rw_in.at[pl.ds(0, rows)], rw_out, sem2)
    c1.start()
    c2.start()
    c1.wait()
    c2.wait()


def kernel(hidden_states, selected_experts, routing_weights):
    rows = hidden_states.shape[0] * hidden_states.shape[1]
    k = selected_experts.shape[1]
    out_dtype = hidden_states.dtype

    se_out, rw_out = pl.pallas_call(
        _copy_body,
        in_specs=[
            pl.BlockSpec(memory_space=pl.ANY),
            pl.BlockSpec(memory_space=pl.ANY),
        ],
        out_specs=[
            pl.BlockSpec(memory_space=pl.ANY),
            pl.BlockSpec(memory_space=pl.ANY),
        ],
        out_shape=[
            jax.ShapeDtypeStruct((rows, k), selected_experts.dtype),
            jax.ShapeDtypeStruct((rows, k), out_dtype),
        ],
        scratch_shapes=[pltpu.SemaphoreType.DMA, pltpu.SemaphoreType.DMA],
    )(selected_experts, routing_weights)

    return se_out, rw_out


# speedup vs baseline: 16.6151x; 16.6151x over previous
import jax
import jax.numpy as jnp
from jax.experimental import pallas as pl
from jax.experimental.pallas import tpu as pltpu


def _copy_body(se_in, rw_in, se_out, rw_out, sem1, sem2):
    rows = se_out.shape[1]
    c1 = pltpu.make_async_copy(se_in.at[:, pl.ds(0, rows)], se_out, sem1)
    c2 = pltpu.make_async_copy(rw_in.at[:, pl.ds(0, rows)], rw_out, sem2)
    c1.start()
    c2.start()
    c1.wait()
    c2.wait()


def kernel(hidden_states, selected_experts, routing_weights):
    rows = hidden_states.shape[0] * hidden_states.shape[1]
    k = selected_experts.shape[1]
    out_dtype = hidden_states.dtype

    se_t = selected_experts.T
    rw_t = routing_weights.astype(out_dtype).T

    se_o, rw_o = pl.pallas_call(
        _copy_body,
        in_specs=[
            pl.BlockSpec(memory_space=pl.ANY),
            pl.BlockSpec(memory_space=pl.ANY),
        ],
        out_specs=[
            pl.BlockSpec(memory_space=pl.ANY),
            pl.BlockSpec(memory_space=pl.ANY),
        ],
        out_shape=[
            jax.ShapeDtypeStruct((k, rows), selected_experts.dtype),
            jax.ShapeDtypeStruct((k, rows), out_dtype),
        ],
        scratch_shapes=[pltpu.SemaphoreType.DMA, pltpu.SemaphoreType.DMA],
    )(se_t, rw_t)

    return se_o.T, rw_o.T


# speedup vs baseline: 245.3981x; 14.7696x over previous
import jax
import jax.numpy as jnp
from jax.experimental import pallas as pl


_GRID = 4


def _copy_body(se_in, rw_in, se_out, rw_out):
    se_out[...] = se_in[...]
    rw_out[...] = rw_in[...]


def kernel(hidden_states, selected_experts, routing_weights):
    rows = hidden_states.shape[0] * hidden_states.shape[1]
    k = selected_experts.shape[1]
    out_dtype = hidden_states.dtype

    se_t = selected_experts.T
    rw_t = routing_weights.astype(out_dtype).T

    block = rows // _GRID
    spec = pl.BlockSpec((k, block), lambda i: (0, i))

    se_o, rw_o = pl.pallas_call(
        _copy_body,
        grid=(_GRID,),
        in_specs=[spec, spec],
        out_specs=[spec, spec],
        out_shape=[
            jax.ShapeDtypeStruct((k, rows), selected_experts.dtype),
            jax.ShapeDtypeStruct((k, rows), out_dtype),
        ],
    )(se_t, rw_t)

    return se_o.T, rw_o.T
